# baseline (speedup 1.0000x reference)
import functools

import jax
import jax.numpy as jnp
from jax import lax
from jax.experimental import pallas as pl
from jax.experimental.pallas import tpu as pltpu
from jax.experimental.pallas import tpu_sc as plsc

G = 200
Gpad = 208
A = 20000
BA = 2048
NB = 10
Ap = BA * NB
NW = 32
CHUNK = Ap // NW
NG = CHUNK // 16
IOU_THRESHOLD = 0.5
BIG = 2**30


def _stage1_body(boxes_ref, anch_ref, iou_out, idx_out, gbest_out,
                 acc_iou, acc_idx, gcol_s):
    j = pl.program_id(0)

    @pl.when(j == 0)
    def _hoist():
        ones = jnp.ones((G, BA), jnp.float32)
        bl = boxes_ref[:, 0:1] * ones
        bt = boxes_ref[:, 1:2] * ones
        br = boxes_ref[:, 2:3] * ones
        bb = boxes_ref[:, 3:4] * ones
        gcol_s[0] = bl
        gcol_s[1] = bt
        gcol_s[2] = br
        gcol_s[3] = bb
        gcol_s[4] = (br - bl) * (bb - bt)

    bl = gcol_s[0]
    bt = gcol_s[1]
    br = gcol_s[2]
    bb = gcol_s[3]
    a1 = gcol_s[4]
    al = anch_ref[0:1, :]
    at = anch_ref[1:2, :]
    ar = anch_ref[2:3, :]
    ab = anch_ref[3:4, :]

    w = jnp.maximum(jnp.minimum(br, ar) - jnp.maximum(bl, al), 0.0)
    h = jnp.maximum(jnp.minimum(bb, ab) - jnp.maximum(bt, at), 0.0)
    inter = w * h
    a2 = (ar - al) * (ab - at)
    iou = inter / (a1 + a2 - inter)

    gi = jax.lax.broadcasted_iota(jnp.int32, (G, BA), 0)
    ai = jax.lax.broadcasted_iota(jnp.int32, (G, BA), 1) + j * BA

    m = jnp.max(iou, axis=0, keepdims=True)
    amin = jnp.min(jnp.where(iou == m, gi, BIG), axis=0, keepdims=True)
    iou_out[0:1, :] = m
    idx_out[0:1, :] = amin

    rmax = jnp.max(iou, axis=1, keepdims=True)
    ridx = jnp.min(jnp.where(iou == rmax, ai, BIG), axis=1, keepdims=True)

    @pl.when(j == 0)
    def _():
        acc_iou[:, 0:1] = jnp.full((G, 1), -1.0, jnp.float32)

    prev_i = acc_iou[:, 0:1]
    upd = rmax > prev_i
    acc_iou[:, 0:1] = jnp.where(upd, rmax, prev_i)

    @pl.when(j == 0)
    def _():
        acc_idx[:, 0:1] = ridx

    @pl.when(j > 0)
    def _():
        acc_idx[:, 0:1] = jnp.where(upd, ridx, acc_idx[:, 0:1])

    @pl.when(j == NB - 1)
    def _():
        gbest_out[:, :] = acc_idx[:, 0:1]


def _make_sc_stage2():
    mesh = plsc.VectorSubcoreMesh(core_axis_name="c", subcore_axis_name="s")

    @functools.partial(
        pl.kernel,
        out_type=[
            jax.ShapeDtypeStruct((Ap, 4), jnp.float32),
            jax.ShapeDtypeStruct((Ap,), jnp.int32),
        ],
        mesh=mesh,
        compiler_params=pltpu.CompilerParams(needs_layout_passes=False),
        scratch_types=[
            pltpu.VMEM((CHUNK,), jnp.float32),
            pltpu.VMEM((CHUNK,), jnp.int32),
            pltpu.VMEM((CHUNK,), jnp.int32),
            pltpu.VMEM((CHUNK,), jnp.float32),
            pltpu.VMEM((CHUNK,), jnp.float32),
            pltpu.VMEM((CHUNK,), jnp.float32),
            pltpu.VMEM((CHUNK,), jnp.float32),
            pltpu.VMEM((Gpad,), jnp.int32),
            pltpu.VMEM((Gpad, 16), jnp.float32),
            pltpu.VMEM((Gpad,), jnp.int32),
            pltpu.VMEM((CHUNK, 4), jnp.float32),
            pltpu.VMEM((CHUNK,), jnp.int32),
        ],
    )
    def sc_stage2(biou_hbm, bidx_hbm, gbest_hbm, table_hbm, labels_hbm,
                  anch_hbm, bbox_hbm, labout_hbm,
                  iou_v, idx_v, forced_v, al_v, at_v, ar_v, ab_v,
                  gb_v, tab_v, lab_v, obox_v, olab_v):
        wid = lax.axis_index("s") * 2 + lax.axis_index("c")
        base = wid * CHUNK

        pltpu.sync_copy(biou_hbm.at[pl.ds(base, CHUNK)], iou_v)
        pltpu.sync_copy(bidx_hbm.at[pl.ds(base, CHUNK)], idx_v)
        pltpu.sync_copy(gbest_hbm, gb_v)
        pltpu.sync_copy(table_hbm, tab_v)
        pltpu.sync_copy(labels_hbm, lab_v)
        pltpu.sync_copy(anch_hbm.at[0, pl.ds(base, CHUNK)], al_v)
        pltpu.sync_copy(anch_hbm.at[1, pl.ds(base, CHUNK)], at_v)
        pltpu.sync_copy(anch_hbm.at[2, pl.ds(base, CHUNK)], ar_v)
        pltpu.sync_copy(anch_hbm.at[3, pl.ds(base, CHUNK)], ab_v)

        for q in range(NG):
            forced_v[pl.ds(q * 16, 16)] = jnp.full((16,), -1, jnp.int32)

        lane = lax.iota(jnp.int32, 16)

        def fbody(k, carry):
            gv = gb_v[pl.ds(k * 16, 16)]
            for ln in range(16):
                off = gv[ln] - base
                offb = jnp.full((16,), off, jnp.int32)
                valb = jnp.full((16,), k * 16 + ln, jnp.int32)
                m = (lane == ln) & (offb >= 0) & (offb < CHUNK)
                offc = jnp.clip(offb, 0, CHUNK - 1)
                plsc.store_scatter(forced_v, [offc], valb, mask=m)
            return carry

        lax.fori_loop(0, Gpad // 16, fbody, 0)

        c0 = jnp.full((16,), 0, jnp.int32)
        for q in range(NG):
            sl = pl.ds(q * 16, 16)
            fg = forced_v[sl]
            fi = jnp.where(fg >= 0, fg, idx_v[sl])
            matched = (fg >= 0) | (iou_v[sl] > IOU_THRESHOLD)
            gx = plsc.load_gather(tab_v, [fi, c0])
            gy = plsc.load_gather(tab_v, [fi, c0 + 1])
            gw = plsc.load_gather(tab_v, [fi, c0 + 2])
            gh = plsc.load_gather(tab_v, [fi, c0 + 3])
            glab = plsc.load_gather(lab_v, [fi])
            al = al_v[sl]
            at = at_v[sl]
            ar = ar_v[sl]
            ab = ab_v[sl]
            x = jnp.where(matched, gx, 0.5 * (al + ar))
            y = jnp.where(matched, gy, 0.5 * (at + ab))
            w = jnp.where(matched, gw, ar - al)
            h = jnp.where(matched, gh, ab - at)
            olab_v[sl] = jnp.where(matched, glab, 0)
            aid = lane + q * 16
            plsc.store_scatter(obox_v, [aid, c0], x)
            plsc.store_scatter(obox_v, [aid, c0 + 1], y)
            plsc.store_scatter(obox_v, [aid, c0 + 2], w)
            plsc.store_scatter(obox_v, [aid, c0 + 3], h)

        pltpu.sync_copy(obox_v, bbox_hbm.at[pl.ds(base, CHUNK)])
        pltpu.sync_copy(olab_v, labout_hbm.at[pl.ds(base, CHUNK)])

    return sc_stage2


@jax.jit
def _run(image, boxes, labels, anchors):
    f32 = jnp.float32
    boxes = boxes.astype(f32)
    anchors = anchors.astype(f32)
    anch_t = jnp.zeros((4, Ap), f32).at[:, :A].set(anchors.T)

    iou_b, idx_b, gbest = pl.pallas_call(
        _stage1_body,
        grid=(NB,),
        in_specs=[
            pl.BlockSpec((G, 4), lambda j: (0, 0)),
            pl.BlockSpec((4, BA), lambda j: (0, j)),
        ],
        out_specs=[
            pl.BlockSpec((1, BA), lambda j: (0, j)),
            pl.BlockSpec((1, BA), lambda j: (0, j)),
            pl.BlockSpec((G, 1), lambda j: (0, 0)),
        ],
        out_shape=[
            jax.ShapeDtypeStruct((1, Ap), f32),
            jax.ShapeDtypeStruct((1, Ap), jnp.int32),
            jax.ShapeDtypeStruct((G, 1), jnp.int32),
        ],
        scratch_shapes=[
            pltpu.VMEM((G, 128), f32),
            pltpu.VMEM((G, 128), jnp.int32),
            pltpu.VMEM((5, G, BA), f32),
        ],
    )(boxes, anch_t)

    x = 0.5 * (boxes[:, 0] + boxes[:, 2])
    y = 0.5 * (boxes[:, 1] + boxes[:, 3])
    w = boxes[:, 2] - boxes[:, 0]
    h = boxes[:, 3] - boxes[:, 1]
    table = (jnp.zeros((Gpad, 16), f32)
             .at[:G, 0].set(x).at[:G, 1].set(y)
             .at[:G, 2].set(w).at[:G, 3].set(h))
    lab_pad = jnp.zeros((Gpad,), jnp.int32).at[:G].set(
        labels.astype(jnp.int32))
    gbest_pad = jnp.full((Gpad,), -1, jnp.int32).at[:G].set(
        gbest.reshape(G))

    bbox, lab = _make_sc_stage2()(
        iou_b.reshape(Ap), idx_b.reshape(Ap), gbest_pad, table, lab_pad,
        anch_t)

    return (image, bbox[:A], lab[:A])


def kernel(image, boxes, labels, anchors):
    return _run(image, boxes, labels, anchors)


# speedup vs baseline: 1.0130x; 1.0130x over previous
import functools

import jax
import jax.numpy as jnp
from jax import lax
from jax.experimental import pallas as pl
from jax.experimental.pallas import tpu as pltpu
from jax.experimental.pallas import tpu_sc as plsc

G = 200
Gpad = 208
A = 20000
BA = 2048
NB = 10
Ap = BA * NB
NW = 32
CHUNK = Ap // NW
NG = CHUNK // 16
IOU_THRESHOLD = 0.5
BIG = 2**30


def _stage1_body(boxes_ref, anch_ref, iou_out, idx_out, gbest_out,
                 acc_iou, acc_idx, gcol_s):
    j = pl.program_id(0)

    @pl.when(j == 0)
    def _hoist():
        ones = jnp.ones((G, BA), jnp.float32)
        bl = boxes_ref[:, 0:1] * ones
        bt = boxes_ref[:, 1:2] * ones
        br = boxes_ref[:, 2:3] * ones
        bb = boxes_ref[:, 3:4] * ones
        gcol_s[0] = bl
        gcol_s[1] = bt
        gcol_s[2] = br
        gcol_s[3] = bb
        gcol_s[4] = (br - bl) * (bb - bt)

    bl = gcol_s[0]
    bt = gcol_s[1]
    br = gcol_s[2]
    bb = gcol_s[3]
    a1 = gcol_s[4]
    al = anch_ref[0:1, :]
    at = anch_ref[1:2, :]
    ar = anch_ref[2:3, :]
    ab = anch_ref[3:4, :]

    w = jnp.maximum(jnp.minimum(br, ar) - jnp.maximum(bl, al), 0.0)
    h = jnp.maximum(jnp.minimum(bb, ab) - jnp.maximum(bt, at), 0.0)
    inter = w * h
    a2 = (ar - al) * (ab - at)
    iou = inter / (a1 + a2 - inter)

    gi = jax.lax.broadcasted_iota(jnp.int32, (G, BA), 0)
    ai = jax.lax.broadcasted_iota(jnp.int32, (G, BA), 1) + j * BA

    m = jnp.max(iou, axis=0, keepdims=True)
    amin = jnp.min(jnp.where(iou == m, gi, BIG), axis=0, keepdims=True)
    iou_out[0:1, :] = m
    idx_out[0:1, :] = amin

    rmax = jnp.max(iou, axis=1, keepdims=True)
    ridx = jnp.min(jnp.where(iou == rmax, ai, BIG), axis=1, keepdims=True)

    @pl.when(j == 0)
    def _():
        acc_iou[:, 0:1] = jnp.full((G, 1), -1.0, jnp.float32)

    prev_i = acc_iou[:, 0:1]
    upd = rmax > prev_i
    acc_iou[:, 0:1] = jnp.where(upd, rmax, prev_i)

    @pl.when(j == 0)
    def _():
        acc_idx[:, 0:1] = ridx

    @pl.when(j > 0)
    def _():
        acc_idx[:, 0:1] = jnp.where(upd, ridx, acc_idx[:, 0:1])

    @pl.when(j == NB - 1)
    def _():
        gbest_out[0:G, :] = acc_idx[:, 0:1]
        gbest_out[G:Gpad, :] = jnp.full((Gpad - G, 1), -1, jnp.int32)


def _make_sc_stage2():
    mesh = plsc.VectorSubcoreMesh(core_axis_name="c", subcore_axis_name="s")

    @functools.partial(
        pl.kernel,
        out_type=[
            jax.ShapeDtypeStruct((Ap, 4), jnp.float32),
            jax.ShapeDtypeStruct((Ap,), jnp.int32),
        ],
        mesh=mesh,
        compiler_params=pltpu.CompilerParams(needs_layout_passes=False),
        scratch_types=[
            pltpu.VMEM((CHUNK,), jnp.float32),
            pltpu.VMEM((CHUNK,), jnp.int32),
            pltpu.VMEM((CHUNK,), jnp.int32),
            pltpu.VMEM((CHUNK * 4,), jnp.float32),
            pltpu.VMEM((Gpad,), jnp.int32),
            pltpu.VMEM((G * 4,), jnp.float32),
            pltpu.VMEM((Gpad,), jnp.float32),
            pltpu.VMEM((Gpad,), jnp.float32),
            pltpu.VMEM((Gpad,), jnp.float32),
            pltpu.VMEM((Gpad,), jnp.float32),
            pltpu.VMEM((G,), jnp.int32),
            pltpu.VMEM((CHUNK, 4), jnp.float32),
            pltpu.VMEM((CHUNK,), jnp.int32),
            pltpu.SemaphoreType.DMA,
        ],
    )
    def sc_stage2(biou_hbm, bidx_hbm, gbest_hbm, boxes_hbm, labels_hbm,
                  anch_hbm, bbox_hbm, labout_hbm,
                  iou_v, idx_v, forced_v, afl_v, gb_v, box_v,
                  tabx_v, taby_v, tabw_v, tabh_v, lab_v,
                  obox_v, olab_v, dsem):
        wid = lax.axis_index("s") * 2 + lax.axis_index("c")
        base = wid * CHUNK

        cps = [
            pltpu.async_copy(biou_hbm.at[pl.ds(base, CHUNK)], iou_v, dsem),
            pltpu.async_copy(bidx_hbm.at[pl.ds(base, CHUNK)], idx_v, dsem),
            pltpu.async_copy(anch_hbm.at[pl.ds(base * 4, CHUNK * 4)],
                             afl_v, dsem),
            pltpu.async_copy(gbest_hbm, gb_v, dsem),
            pltpu.async_copy(boxes_hbm, box_v, dsem),
            pltpu.async_copy(labels_hbm, lab_v, dsem),
        ]
        for cp in cps:
            cp.wait()

        lane = lax.iota(jnp.int32, 16)

        for k in range(Gpad // 16):
            gidx = jnp.minimum(lane + k * 16, G - 1) * 4
            l = plsc.load_gather(box_v, [gidx])
            t = plsc.load_gather(box_v, [gidx + 1])
            r = plsc.load_gather(box_v, [gidx + 2])
            b = plsc.load_gather(box_v, [gidx + 3])
            sl = pl.ds(k * 16, 16)
            tabx_v[sl] = 0.5 * (l + r)
            taby_v[sl] = 0.5 * (t + b)
            tabw_v[sl] = r - l
            tabh_v[sl] = b - t

        for q in range(NG):
            forced_v[pl.ds(q * 16, 16)] = jnp.full((16,), -1, jnp.int32)

        def fbody(k, carry):
            gv = gb_v[pl.ds(k * 16, 16)]
            for ln in range(16):
                off = gv[ln] - base
                offb = jnp.full((16,), off, jnp.int32)
                valb = jnp.full((16,), k * 16 + ln, jnp.int32)
                m = (lane == ln) & (offb >= 0) & (offb < CHUNK)
                offc = jnp.clip(offb, 0, CHUNK - 1)
                plsc.store_scatter(forced_v, [offc], valb, mask=m)
            return carry

        lax.fori_loop(0, Gpad // 16, fbody, 0)

        c0 = jnp.full((16,), 0, jnp.int32)
        for q in range(NG):
            sl = pl.ds(q * 16, 16)
            fg = forced_v[sl]
            fi = jnp.where(fg >= 0, fg, idx_v[sl])
            matched = (fg >= 0) | (iou_v[sl] > IOU_THRESHOLD)
            gx = plsc.load_gather(tabx_v, [fi])
            gy = plsc.load_gather(taby_v, [fi])
            gw = plsc.load_gather(tabw_v, [fi])
            gh = plsc.load_gather(tabh_v, [fi])
            glab = plsc.load_gather(lab_v, [fi])
            apos = (lane + q * 16) * 4
            al = plsc.load_gather(afl_v, [apos])
            at = plsc.load_gather(afl_v, [apos + 1])
            ar = plsc.load_gather(afl_v, [apos + 2])
            ab = plsc.load_gather(afl_v, [apos + 3])
            x = jnp.where(matched, gx, 0.5 * (al + ar))
            y = jnp.where(matched, gy, 0.5 * (at + ab))
            w = jnp.where(matched, gw, ar - al)
            h = jnp.where(matched, gh, ab - at)
            olab_v[sl] = jnp.where(matched, glab, 0)
            aid = lane + q * 16
            plsc.store_scatter(obox_v, [aid, c0], x)
            plsc.store_scatter(obox_v, [aid, c0 + 1], y)
            plsc.store_scatter(obox_v, [aid, c0 + 2], w)
            plsc.store_scatter(obox_v, [aid, c0 + 3], h)

        pltpu.sync_copy(obox_v, bbox_hbm.at[pl.ds(base, CHUNK)])
        pltpu.sync_copy(olab_v, labout_hbm.at[pl.ds(base, CHUNK)])

    return sc_stage2


@jax.jit
def _run(image, boxes, labels, anchors):
    f32 = jnp.float32
    boxes = boxes.astype(f32)
    anchors = anchors.astype(f32)
    anch_pad = jnp.zeros((Ap, 4), f32).at[:A].set(anchors)
    anch_t = anch_pad.T

    iou_b, idx_b, gbest = pl.pallas_call(
        _stage1_body,
        grid=(NB,),
        in_specs=[
            pl.BlockSpec((G, 4), lambda j: (0, 0)),
            pl.BlockSpec((4, BA), lambda j: (0, j)),
        ],
        out_specs=[
            pl.BlockSpec((1, BA), lambda j: (0, j)),
            pl.BlockSpec((1, BA), lambda j: (0, j)),
            pl.BlockSpec((Gpad, 1), lambda j: (0, 0)),
        ],
        out_shape=[
            jax.ShapeDtypeStruct((1, Ap), f32),
            jax.ShapeDtypeStruct((1, Ap), jnp.int32),
            jax.ShapeDtypeStruct((Gpad, 1), jnp.int32),
        ],
        scratch_shapes=[
            pltpu.VMEM((G, 128), f32),
            pltpu.VMEM((G, 128), jnp.int32),
            pltpu.VMEM((5, G, BA), f32),
        ],
    )(boxes, anch_t)

    bbox, lab = _make_sc_stage2()(
        iou_b.reshape(Ap), idx_b.reshape(Ap), gbest.reshape(Gpad),
        boxes.reshape(G * 4), labels.astype(jnp.int32),
        anch_pad.reshape(Ap * 4))

    return (image, bbox[:A], lab[:A])


def kernel(image, boxes, labels, anchors):
    return _run(image, boxes, labels, anchors)
